# baseline (speedup 1.0000x reference)
import functools

import jax
import jax.numpy as jnp
from jax import lax
from jax.experimental import pallas as pl
from jax.experimental.pallas import tpu as pltpu
from jax.experimental.pallas import tpu_sc as plsc

N = 1_600_000
B = 32
K = 64
NSEG = B * K
ROW = 2 * NSEG
NW = 32
ROWS_PER_W = N // NW
CHUNK = 2000
NCHUNK = ROWS_PER_W // CHUNK
VPC = CHUNK // 16

_mesh = plsc.VectorSubcoreMesh(core_axis_name="c", subcore_axis_name="s")


@functools.partial(
    pl.kernel,
    mesh=_mesh,
    out_type=jax.ShapeDtypeStruct((NW, ROW), jnp.float32),
    compiler_params=pltpu.CompilerParams(
        needs_layout_passes=False, use_tc_tiling_on_sc=False
    ),
    scratch_types=[
        pltpu.VMEM((CHUNK, 1), jnp.float32),
        pltpu.VMEM((CHUNK, 5), jnp.float32),
        pltpu.VMEM((CHUNK, 6), jnp.int32),
        pltpu.VMEM((16 * ROW,), jnp.float32),
    ],
)
def _seg_reduce(reco_h, inp_h, cl_h, out_h, reco_v, inp_v, cl_v, acc):
    c = lax.axis_index("c")
    s = lax.axis_index("s")
    wid = c * 16 + s
    base = wid * ROWS_PER_W

    zeros = jnp.zeros((16,), jnp.float32)
    ones = jnp.ones((16,), jnp.float32)
    lane = lax.iota(jnp.int32, 16)
    lane_base = lane * ROW
    col0 = jnp.zeros((16,), jnp.int32)
    col3 = jnp.full((16,), 3, jnp.int32)
    col4 = jnp.full((16,), 4, jnp.int32)

    def zbody(i, carry):
        acc[pl.ds(pl.multiple_of(i * 16, 16), 16)] = zeros
        return carry

    lax.fori_loop(0, ROW, zbody, 0)

    def chunk_body(ch, carry):
        r0 = base + ch * CHUNK
        pltpu.sync_copy(reco_h.at[pl.ds(r0, CHUNK)], reco_v)
        pltpu.sync_copy(inp_h.at[pl.ds(r0, CHUNK)], inp_v)
        pltpu.sync_copy(cl_h.at[pl.ds(r0, CHUNK)], cl_v)

        def vbody(i, carry2):
            lanes = lane + i * 16
            r = plsc.load_gather(reco_v, [lanes, col0])
            t = plsc.load_gather(inp_v, [lanes, col4])
            bb = plsc.load_gather(cl_v, [lanes, col3])
            cc = plsc.load_gather(cl_v, [lanes, col4])
            seg = bb * K + cc
            d = r - t
            idx = lane_base + seg
            plsc.addupdate_scatter(acc, [idx], d * d)
            plsc.addupdate_scatter(acc, [idx + NSEG], ones)
            return carry2

        lax.fori_loop(0, VPC, vbody, 0)
        return carry

    lax.fori_loop(0, NCHUNK, chunk_body, 0)

    def rbody(j, carry):
        p = pl.multiple_of(j * 16, 16)
        def lbody(l, v):
            return v + acc[pl.ds(l * ROW + p, 16)]
        acc[pl.ds(p, 16)] = lax.fori_loop(1, 16, lbody, acc[pl.ds(p, 16)])
        return carry

    lax.fori_loop(0, ROW // 16, rbody, 0)

    pltpu.sync_copy(acc.at[pl.ds(0, ROW)], out_h.at[wid])


def _epilogue(p_ref, a_ref, o_ref):
    p = p_ref[...]
    s = jnp.sum(p[:, :NSEG], axis=0, keepdims=True)
    cnt = jnp.sum(p[:, NSEG:], axis=0, keepdims=True)
    pres = cnt > 0.0
    mse = jnp.where(pres, s / jnp.maximum(cnt, 1.0), 0.0)
    a = a_ref[...]
    bsum = jnp.dot(mse, a, preferred_element_type=jnp.float32,
                   precision=lax.Precision.HIGHEST)
    ncl = jnp.dot(pres.astype(jnp.float32), a,
                  preferred_element_type=jnp.float32,
                  precision=lax.Precision.HIGHEST)
    bl = bsum / jnp.maximum(ncl, 1.0)
    bp = ncl > 0.0
    loss = jnp.sum(jnp.where(bp, bl, 0.0)) / jnp.maximum(
        jnp.sum(bp.astype(jnp.float32)), 1.0)
    o_ref[...] = jnp.full((1, 1), loss, jnp.float32)


def kernel(reco, input_data0, cluster_label0):
    parts = _seg_reduce(reco, input_data0, cluster_label0)
    a = (jnp.arange(NSEG, dtype=jnp.int32)[:, None] // K
         == jnp.arange(B, dtype=jnp.int32)[None, :]).astype(jnp.float32)
    loss2d = pl.pallas_call(
        _epilogue,
        out_shape=jax.ShapeDtypeStruct((1, 1), jnp.float32),
    )(parts, a)
    return loss2d[0, 0]


# speedup vs baseline: 1.5229x; 1.5229x over previous
import functools

import jax
import jax.numpy as jnp
from jax import lax
from jax.experimental import pallas as pl
from jax.experimental.pallas import tpu as pltpu
from jax.experimental.pallas import tpu_sc as plsc

N = 1_600_000
B = 32
K = 64
NSEG = B * K
ROW = 2 * NSEG
NW = 32
ROWS_PER_W = N // NW
CHUNK = 10000
NCHUNK = ROWS_PER_W // CHUNK
VPC = CHUNK // 16

PBLK = 8192
PGRID = -(-N // PBLK)


def _prepass(reco_ref, inp_ref, cl_ref, sq_ref, seg_ref):
    inp = inp_ref[...]
    cl = cl_ref[...]
    rc = reco_ref[...]
    c5 = lax.broadcasted_iota(jnp.int32, (1, 5), 1)
    c6 = lax.broadcasted_iota(jnp.int32, (1, 6), 1)
    t = jnp.sum(jnp.where(c5 == 4, inp, 0.0), axis=1)
    r = jnp.sum(rc, axis=1)
    d = r - t
    bb = jnp.sum(jnp.where(c6 == 3, cl, 0), axis=1)
    cc = jnp.sum(jnp.where(c6 == 4, cl, 0), axis=1)
    sq_ref[...] = d * d
    seg_ref[...] = bb * K + cc


_mesh = plsc.VectorSubcoreMesh(core_axis_name="c", subcore_axis_name="s")


@functools.partial(
    pl.kernel,
    mesh=_mesh,
    out_type=jax.ShapeDtypeStruct((NW, ROW), jnp.float32),
    compiler_params=pltpu.CompilerParams(needs_layout_passes=False),
    scratch_types=[
        pltpu.VMEM((CHUNK,), jnp.float32),
        pltpu.VMEM((CHUNK,), jnp.int32),
        pltpu.VMEM((16 * ROW,), jnp.float32),
    ],
)
def _seg_reduce(sq_h, seg_h, out_h, sq_v, seg_v, acc):
    c = lax.axis_index("c")
    s = lax.axis_index("s")
    wid = c * 16 + s
    base = wid * ROWS_PER_W

    zeros = jnp.zeros((16,), jnp.float32)
    ones = jnp.ones((16,), jnp.float32)
    lane = lax.iota(jnp.int32, 16)
    lane_base = lane * ROW

    def zbody(i, carry):
        acc[pl.ds(pl.multiple_of(i * 16, 16), 16)] = zeros
        return carry

    lax.fori_loop(0, ROW, zbody, 0)

    def chunk_body(ch, carry):
        r0 = base + ch * CHUNK
        pltpu.sync_copy(sq_h.at[pl.ds(r0, CHUNK)], sq_v)
        pltpu.sync_copy(seg_h.at[pl.ds(r0, CHUNK)], seg_v)

        def vbody(i, carry2):
            p = pl.multiple_of(i * 16, 16)
            sq = sq_v[pl.ds(p, 16)]
            seg = seg_v[pl.ds(p, 16)]
            idx = lane_base + seg
            plsc.addupdate_scatter(acc, [idx], sq)
            plsc.addupdate_scatter(acc, [idx + NSEG], ones)
            return carry2

        lax.fori_loop(0, VPC, vbody, 0)
        return carry

    lax.fori_loop(0, NCHUNK, chunk_body, 0)

    def rbody(j, carry):
        p = pl.multiple_of(j * 16, 16)
        def lbody(l, v):
            return v + acc[pl.ds(l * ROW + p, 16)]
        acc[pl.ds(p, 16)] = lax.fori_loop(1, 16, lbody, acc[pl.ds(p, 16)])
        return carry

    lax.fori_loop(0, ROW // 16, rbody, 0)

    pltpu.sync_copy(acc.at[pl.ds(0, ROW)], out_h.at[wid])


def _epilogue(p_ref, a_ref, o_ref):
    p = p_ref[...]
    s = jnp.sum(p[:, :NSEG], axis=0, keepdims=True)
    cnt = jnp.sum(p[:, NSEG:], axis=0, keepdims=True)
    pres = cnt > 0.0
    mse = jnp.where(pres, s / jnp.maximum(cnt, 1.0), 0.0)
    a = a_ref[...]
    bsum = jnp.dot(mse, a, preferred_element_type=jnp.float32,
                   precision=lax.Precision.HIGHEST)
    ncl = jnp.dot(pres.astype(jnp.float32), a,
                  preferred_element_type=jnp.float32,
                  precision=lax.Precision.HIGHEST)
    bl = bsum / jnp.maximum(ncl, 1.0)
    bp = ncl > 0.0
    loss = jnp.sum(jnp.where(bp, bl, 0.0)) / jnp.maximum(
        jnp.sum(bp.astype(jnp.float32)), 1.0)
    o_ref[...] = jnp.full((1, 1), loss, jnp.float32)


def kernel(reco, input_data0, cluster_label0):
    sq, seg = pl.pallas_call(
        _prepass,
        grid=(PGRID,),
        in_specs=[
            pl.BlockSpec((PBLK, 1), lambda i: (i, 0)),
            pl.BlockSpec((PBLK, 5), lambda i: (i, 0)),
            pl.BlockSpec((PBLK, 6), lambda i: (i, 0)),
        ],
        out_specs=[
            pl.BlockSpec((PBLK,), lambda i: (i,)),
            pl.BlockSpec((PBLK,), lambda i: (i,)),
        ],
        out_shape=[
            jax.ShapeDtypeStruct((N,), jnp.float32),
            jax.ShapeDtypeStruct((N,), jnp.int32),
        ],
    )(reco, input_data0, cluster_label0)

    parts = _seg_reduce(sq, seg)

    a = (jnp.arange(NSEG, dtype=jnp.int32)[:, None] // K
         == jnp.arange(B, dtype=jnp.int32)[None, :]).astype(jnp.float32)
    loss2d = pl.pallas_call(
        _epilogue,
        out_shape=jax.ShapeDtypeStruct((1, 1), jnp.float32),
    )(parts, a)
    return loss2d[0, 0]


# speedup vs baseline: 24.6832x; 16.2075x over previous
import functools

import jax
import jax.numpy as jnp
from jax import lax
from jax.experimental import pallas as pl
from jax.experimental.pallas import tpu as pltpu
from jax.experimental.pallas import tpu_sc as plsc

N = 1_600_000
B = 32
K = 64
NSEG = B * K
ROW = 2 * NSEG
NW = 32
ROWS_PER_W = N // NW
CHUNK = 10000
NCHUNK = ROWS_PER_W // CHUNK
VPC = CHUNK // 16

PBLK = 8192
PGRID = -(-N // PBLK)


def _prepass(reco_ref, inp_ref, cl_ref, sq_ref, seg_ref):
    t = inp_ref[4:5, :]
    r = reco_ref[...]
    d = r - t
    bb = cl_ref[3:4, :]
    cc = cl_ref[4:5, :]
    sq_ref[...] = jnp.reshape(d * d, (PBLK,))
    seg_ref[...] = jnp.reshape(bb * K + cc, (PBLK,))


_mesh = plsc.VectorSubcoreMesh(core_axis_name="c", subcore_axis_name="s")


@functools.partial(
    pl.kernel,
    mesh=_mesh,
    out_type=jax.ShapeDtypeStruct((NW, ROW), jnp.float32),
    compiler_params=pltpu.CompilerParams(needs_layout_passes=False),
    scratch_types=[
        pltpu.VMEM((CHUNK,), jnp.float32),
        pltpu.VMEM((CHUNK,), jnp.int32),
        pltpu.VMEM((16 * ROW,), jnp.float32),
    ],
)
def _seg_reduce(sq_h, seg_h, out_h, sq_v, seg_v, acc):
    c = lax.axis_index("c")
    s = lax.axis_index("s")
    wid = c * 16 + s
    base = wid * ROWS_PER_W

    zeros = jnp.zeros((16,), jnp.float32)
    ones = jnp.ones((16,), jnp.float32)
    lane = lax.iota(jnp.int32, 16)
    lane_base = lane * ROW

    def zbody(i, carry):
        acc[pl.ds(pl.multiple_of(i * 16, 16), 16)] = zeros
        return carry

    lax.fori_loop(0, ROW, zbody, 0)

    def chunk_body(ch, carry):
        r0 = base + ch * CHUNK
        pltpu.sync_copy(sq_h.at[pl.ds(r0, CHUNK)], sq_v)
        pltpu.sync_copy(seg_h.at[pl.ds(r0, CHUNK)], seg_v)

        def vbody(i, carry2):
            p = pl.multiple_of(i * 16, 16)
            sq = sq_v[pl.ds(p, 16)]
            seg = seg_v[pl.ds(p, 16)]
            idx = lane_base + seg
            plsc.addupdate_scatter(acc, [idx], sq)
            plsc.addupdate_scatter(acc, [idx + NSEG], ones)
            return carry2

        lax.fori_loop(0, VPC, vbody, 0)
        return carry

    lax.fori_loop(0, NCHUNK, chunk_body, 0)

    def rbody(j, carry):
        p = pl.multiple_of(j * 16, 16)
        def lbody(l, v):
            return v + acc[pl.ds(l * ROW + p, 16)]
        acc[pl.ds(p, 16)] = lax.fori_loop(1, 16, lbody, acc[pl.ds(p, 16)])
        return carry

    lax.fori_loop(0, ROW // 16, rbody, 0)

    pltpu.sync_copy(acc.at[pl.ds(0, ROW)], out_h.at[wid])


def _epilogue(p_ref, a_ref, o_ref):
    p = p_ref[...]
    s = jnp.sum(p[:, :NSEG], axis=0, keepdims=True)
    cnt = jnp.sum(p[:, NSEG:], axis=0, keepdims=True)
    pres = cnt > 0.0
    mse = jnp.where(pres, s / jnp.maximum(cnt, 1.0), 0.0)
    a = a_ref[...]
    bsum = jnp.dot(mse, a, preferred_element_type=jnp.float32,
                   precision=lax.Precision.HIGHEST)
    ncl = jnp.dot(pres.astype(jnp.float32), a,
                  preferred_element_type=jnp.float32,
                  precision=lax.Precision.HIGHEST)
    bl = bsum / jnp.maximum(ncl, 1.0)
    bp = ncl > 0.0
    loss = jnp.sum(jnp.where(bp, bl, 0.0)) / jnp.maximum(
        jnp.sum(bp.astype(jnp.float32)), 1.0)
    o_ref[...] = jnp.full((1, 1), loss, jnp.float32)


def kernel(reco, input_data0, cluster_label0):
    sq, seg = pl.pallas_call(
        _prepass,
        grid=(PGRID,),
        in_specs=[
            pl.BlockSpec((1, PBLK), lambda i: (0, i)),
            pl.BlockSpec((5, PBLK), lambda i: (0, i)),
            pl.BlockSpec((6, PBLK), lambda i: (0, i)),
        ],
        out_specs=[
            pl.BlockSpec((PBLK,), lambda i: (i,)),
            pl.BlockSpec((PBLK,), lambda i: (i,)),
        ],
        out_shape=[
            jax.ShapeDtypeStruct((N,), jnp.float32),
            jax.ShapeDtypeStruct((N,), jnp.int32),
        ],
    )(reco.T, input_data0.T, cluster_label0.T)

    parts = _seg_reduce(sq, seg)

    a = (jnp.arange(NSEG, dtype=jnp.int32)[:, None] // K
         == jnp.arange(B, dtype=jnp.int32)[None, :]).astype(jnp.float32)
    loss2d = pl.pallas_call(
        _epilogue,
        out_shape=jax.ShapeDtypeStruct((1, 1), jnp.float32),
    )(parts, a)
    return loss2d[0, 0]


# speedup vs baseline: 46.1740x; 1.8707x over previous
import functools

import jax
import jax.numpy as jnp
from jax import lax
from jax.experimental import pallas as pl
from jax.experimental.pallas import tpu as pltpu
from jax.experimental.pallas import tpu_sc as plsc

N = 1_600_000
B = 32
K = 64
NSEG = B * K
ROW = 2 * NSEG
NW = 32
ROWS_PER_W = N // NW
CHUNK = 10000
NCHUNK = ROWS_PER_W // CHUNK
VPC = CHUNK // 16

PBLK = 32768
PGRID = -(-N // PBLK)


def _prepass(reco_ref, inp_ref, cl_ref, sq_ref, seg_ref):
    d = reco_ref[...] - inp_ref[4:5, :]
    seg = cl_ref[3:4, :] * K + cl_ref[4:5, :]
    sq_ref[...] = jnp.reshape(d * d, (PBLK,))
    seg_ref[...] = jnp.reshape(seg, (PBLK,))


_mesh = plsc.VectorSubcoreMesh(core_axis_name="c", subcore_axis_name="s")


@functools.partial(
    pl.kernel,
    mesh=_mesh,
    out_type=jax.ShapeDtypeStruct((NW, ROW), jnp.float32),
    compiler_params=pltpu.CompilerParams(needs_layout_passes=False),
    scratch_types=[
        pltpu.VMEM((CHUNK,), jnp.float32),
        pltpu.VMEM((CHUNK,), jnp.float32),
        pltpu.VMEM((CHUNK,), jnp.int32),
        pltpu.VMEM((CHUNK,), jnp.int32),
        pltpu.VMEM((16 * ROW,), jnp.float32),
        pltpu.SemaphoreType.DMA,
        pltpu.SemaphoreType.DMA,
    ],
)
def _seg_reduce(sq_h, seg_h, out_h, sq_v0, sq_v1, seg_v0, seg_v1, acc,
                sem0, sem1):
    c = lax.axis_index("c")
    s = lax.axis_index("s")
    wid = c * 16 + s
    base = wid * ROWS_PER_W

    zeros = jnp.zeros((16,), jnp.float32)
    ones = jnp.ones((16,), jnp.float32)
    lane = lax.iota(jnp.int32, 16)
    lane_base = lane * ROW
    sems = (sem0, sem1)
    sq_bufs = (sq_v0, sq_v1)
    seg_bufs = (seg_v0, seg_v1)

    def zbody(i, carry):
        for u in range(4):
            acc[pl.ds(pl.multiple_of((i * 4 + u) * 16, 16), 16)] = zeros
        return carry

    lax.fori_loop(0, ROW // 4, zbody, 0)

    def start(ch):
        b = ch % 2
        r0 = base + ch * CHUNK
        h1 = pltpu.async_copy(sq_h.at[pl.ds(r0, CHUNK)], sq_bufs[b], sems[b])
        h2 = pltpu.async_copy(seg_h.at[pl.ds(r0, CHUNK)], seg_bufs[b], sems[b])
        return h1, h2

    hs = start(0)
    for ch in range(NCHUNK):
        h1, h2 = hs
        h1.wait()
        h2.wait()
        if ch + 1 < NCHUNK:
            hs = start(ch + 1)
        sq_v = sq_bufs[ch % 2]
        seg_v = seg_bufs[ch % 2]

        def vbody(i, carry2):
            for u in range(5):
                p = pl.multiple_of(i * 80 + u * 16, 16)
                sq = sq_v[pl.ds(p, 16)]
                seg = seg_v[pl.ds(p, 16)]
                idx = lane_base + seg
                plsc.addupdate_scatter(acc, [idx], sq)
                plsc.addupdate_scatter(acc, [idx + NSEG], ones)
            return carry2

        lax.fori_loop(0, VPC // 5, vbody, 0)

    def rbody(j, carry):
        p = pl.multiple_of(j * 16, 16)
        v = acc[pl.ds(p, 16)]
        for l in range(1, 16):
            v = v + acc[pl.ds(l * ROW + p, 16)]
        acc[pl.ds(p, 16)] = v
        return carry

    lax.fori_loop(0, ROW // 16, rbody, 0)

    pltpu.sync_copy(acc.at[pl.ds(0, ROW)], out_h.at[wid])


def _epilogue(p_ref, a_ref, o_ref):
    p = p_ref[...]
    s = jnp.sum(p[:, :NSEG], axis=0, keepdims=True)
    cnt = jnp.sum(p[:, NSEG:], axis=0, keepdims=True)
    pres = cnt > 0.0
    mse = jnp.where(pres, s / jnp.maximum(cnt, 1.0), 0.0)
    a = a_ref[...]
    bsum = jnp.dot(mse, a, preferred_element_type=jnp.float32,
                   precision=lax.Precision.HIGHEST)
    ncl = jnp.dot(pres.astype(jnp.float32), a,
                  preferred_element_type=jnp.float32,
                  precision=lax.Precision.HIGHEST)
    bl = bsum / jnp.maximum(ncl, 1.0)
    bp = ncl > 0.0
    loss = jnp.sum(jnp.where(bp, bl, 0.0)) / jnp.maximum(
        jnp.sum(bp.astype(jnp.float32)), 1.0)
    o_ref[...] = jnp.full((1, 1), loss, jnp.float32)


def kernel(reco, input_data0, cluster_label0):
    sq, seg = pl.pallas_call(
        _prepass,
        grid=(PGRID,),
        in_specs=[
            pl.BlockSpec((1, PBLK), lambda i: (0, i)),
            pl.BlockSpec((5, PBLK), lambda i: (0, i)),
            pl.BlockSpec((6, PBLK), lambda i: (0, i)),
        ],
        out_specs=[
            pl.BlockSpec((PBLK,), lambda i: (i,)),
            pl.BlockSpec((PBLK,), lambda i: (i,)),
        ],
        out_shape=[
            jax.ShapeDtypeStruct((N,), jnp.float32),
            jax.ShapeDtypeStruct((N,), jnp.int32),
        ],
    )(reco.T, input_data0.T, cluster_label0.T)

    parts = _seg_reduce(sq, seg)

    a = (jnp.arange(NSEG, dtype=jnp.int32)[:, None] // K
         == jnp.arange(B, dtype=jnp.int32)[None, :]).astype(jnp.float32)
    loss2d = pl.pallas_call(
        _epilogue,
        out_shape=jax.ShapeDtypeStruct((1, 1), jnp.float32),
    )(parts, a)
    return loss2d[0, 0]


# speedup vs baseline: 51.3399x; 1.1119x over previous
import functools

import jax
import jax.numpy as jnp
from jax import lax
from jax.experimental import pallas as pl
from jax.experimental.pallas import tpu as pltpu
from jax.experimental.pallas import tpu_sc as plsc

N = 1_600_000
B = 32
K = 64
NSEG = B * K
ROW = 2 * NSEG
NW = 32
ROWS_PER_W = N // NW
CHUNK = 10000
NCHUNK = ROWS_PER_W // CHUNK
VPC = CHUNK // 16

PBLK = 65536
PGRID = -(-N // PBLK)


def _prepass(reco_ref, inp_ref, cl_ref, sq_ref, seg_ref):
    d = reco_ref[...] - inp_ref[4:5, :]
    seg = cl_ref[3:4, :] * K + cl_ref[4:5, :]
    sq_ref[...] = jnp.reshape(d * d, (PBLK,))
    seg_ref[...] = jnp.reshape(seg, (PBLK,))


_mesh = plsc.VectorSubcoreMesh(core_axis_name="c", subcore_axis_name="s")


@functools.partial(
    pl.kernel,
    mesh=_mesh,
    out_type=jax.ShapeDtypeStruct((NW, ROW), jnp.float32),
    compiler_params=pltpu.CompilerParams(needs_layout_passes=False),
    scratch_types=[
        pltpu.VMEM((CHUNK,), jnp.float32),
        pltpu.VMEM((CHUNK,), jnp.float32),
        pltpu.VMEM((CHUNK,), jnp.int32),
        pltpu.VMEM((CHUNK,), jnp.int32),
        pltpu.VMEM((16 * ROW,), jnp.float32),
        pltpu.SemaphoreType.DMA,
        pltpu.SemaphoreType.DMA,
    ],
)
def _seg_reduce(sq_h, seg_h, out_h, sq_v0, sq_v1, seg_v0, seg_v1, acc,
                sem0, sem1):
    c = lax.axis_index("c")
    s = lax.axis_index("s")
    wid = c * 16 + s
    base = wid * ROWS_PER_W

    zeros = jnp.zeros((16,), jnp.float32)
    ones = jnp.ones((16,), jnp.float32)
    lane = lax.iota(jnp.int32, 16)
    lane_base = lane * ROW
    sems = (sem0, sem1)
    sq_bufs = (sq_v0, sq_v1)
    seg_bufs = (seg_v0, seg_v1)

    def zbody(i, carry):
        for u in range(4):
            acc[pl.ds(pl.multiple_of((i * 4 + u) * 16, 16), 16)] = zeros
        return carry

    lax.fori_loop(0, ROW // 4, zbody, 0)

    def start(ch):
        b = ch % 2
        r0 = base + ch * CHUNK
        h1 = pltpu.async_copy(sq_h.at[pl.ds(r0, CHUNK)], sq_bufs[b], sems[b])
        h2 = pltpu.async_copy(seg_h.at[pl.ds(r0, CHUNK)], seg_bufs[b], sems[b])
        return h1, h2

    hs = start(0)
    for ch in range(NCHUNK):
        h1, h2 = hs
        h1.wait()
        h2.wait()
        if ch + 1 < NCHUNK:
            hs = start(ch + 1)
        sq_v = sq_bufs[ch % 2]
        seg_v = seg_bufs[ch % 2]

        def vbody(i, carry2):
            for u in range(25):
                p = pl.multiple_of(i * 400 + u * 16, 16)
                sq = sq_v[pl.ds(p, 16)]
                seg = seg_v[pl.ds(p, 16)]
                idx = lane_base + seg
                plsc.addupdate_scatter(acc, [idx], sq)
                plsc.addupdate_scatter(acc, [idx + NSEG], ones)
            return carry2

        lax.fori_loop(0, VPC // 25, vbody, 0)

    def rbody(j, carry):
        p = pl.multiple_of(j * 16, 16)
        v = acc[pl.ds(p, 16)]
        for l in range(1, 16):
            v = v + acc[pl.ds(l * ROW + p, 16)]
        acc[pl.ds(p, 16)] = v
        return carry

    lax.fori_loop(0, ROW // 16, rbody, 0)

    pltpu.sync_copy(acc.at[pl.ds(0, ROW)], out_h.at[wid])


def _epilogue(p_ref, a_ref, o_ref):
    p = p_ref[...]
    s = jnp.sum(p[:, :NSEG], axis=0, keepdims=True)
    cnt = jnp.sum(p[:, NSEG:], axis=0, keepdims=True)
    pres = cnt > 0.0
    mse = jnp.where(pres, s / jnp.maximum(cnt, 1.0), 0.0)
    a = a_ref[...]
    bsum = jnp.dot(mse, a, preferred_element_type=jnp.float32,
                   precision=lax.Precision.HIGHEST)
    ncl = jnp.dot(pres.astype(jnp.float32), a,
                  preferred_element_type=jnp.float32,
                  precision=lax.Precision.HIGHEST)
    bl = bsum / jnp.maximum(ncl, 1.0)
    bp = ncl > 0.0
    loss = jnp.sum(jnp.where(bp, bl, 0.0)) / jnp.maximum(
        jnp.sum(bp.astype(jnp.float32)), 1.0)
    o_ref[...] = jnp.full((1, 1), loss, jnp.float32)


def kernel(reco, input_data0, cluster_label0):
    sq, seg = pl.pallas_call(
        _prepass,
        grid=(PGRID,),
        in_specs=[
            pl.BlockSpec((1, PBLK), lambda i: (0, i)),
            pl.BlockSpec((5, PBLK), lambda i: (0, i)),
            pl.BlockSpec((6, PBLK), lambda i: (0, i)),
        ],
        out_specs=[
            pl.BlockSpec((PBLK,), lambda i: (i,)),
            pl.BlockSpec((PBLK,), lambda i: (i,)),
        ],
        out_shape=[
            jax.ShapeDtypeStruct((N,), jnp.float32),
            jax.ShapeDtypeStruct((N,), jnp.int32),
        ],
    )(reco.T, input_data0.T, cluster_label0.T)

    parts = _seg_reduce(sq, seg)

    a = (jnp.arange(NSEG, dtype=jnp.int32)[:, None] // K
         == jnp.arange(B, dtype=jnp.int32)[None, :]).astype(jnp.float32)
    loss2d = pl.pallas_call(
        _epilogue,
        out_shape=jax.ShapeDtypeStruct((1, 1), jnp.float32),
    )(parts, a)
    return loss2d[0, 0]
